# Initial kernel scaffold; baseline (speedup 1.0000x reference)
import jax
import jax.numpy as jnp
from jax import lax
from jax.experimental import pallas as pl
from jax.experimental.pallas import tpu as pltpu
from jax.experimental.pallas import tpu_sc as plsc

N = 10000
E = 160000
F = 128
NT = 16
EPT_B = E // NT
EPT_D = E // 32
ACC_ROWS = 10240
CNT_LEN = 81920
LISTCAP = EPT_D + 16


def _mm_bias_body(x_ref, w_ref, b_ref, o_ref):
    o_ref[...] = (
        jnp.dot(x_ref[...], w_ref[...], preferred_element_type=jnp.float32)
        + b_ref[...]
    )


def _fused_c_body(y0_ref, agg_ref, ws_ref, wn_ref, bc_ref, w01_ref, b01_ref, o_ref):
    c1 = (
        jnp.dot(y0_ref[...], ws_ref[...], preferred_element_type=jnp.float32)
        + jnp.dot(agg_ref[...], wn_ref[...], preferred_element_type=jnp.float32)
        + bc_ref[...]
    )
    o_ref[...] = (
        jnp.dot(c1, w01_ref[...], preferred_element_type=jnp.float32) + b01_ref[...]
    )


def _final_body(grows_ref, a1p_ref, cn_ref, cs_ref, al_ref, eq_ref,
                w10_ref, b10_ref, ws0_ref, wn0_ref, bc0_ref, w01_ref, b01_ref,
                w11_ref, b11_ref, ws1_ref, wn1_ref, bc1_ref, o_ref):
    dot = lambda a, b: jnp.dot(a, b, preferred_element_type=jnp.float32)
    g = grows_ref[...]
    xs, y0s, aggs, y1s, y1n = g[0:8], g[16:24], g[32:40], g[48:56], g[64:72]
    a1 = a1p_ref[0] + a1p_ref[1]
    cn, cs, al, eq = cn_ref[...], cs_ref[...], al_ref[...], eq_ref[...]
    d = dot(xs, w10_ref[...]) + b10_ref[...] - y0s
    ws = dot(d, ws0_ref[...])
    wn = dot(d, wn0_ref[...])
    c1s = dot(y0s, ws0_ref[...]) + dot(aggs, wn0_ref[...]) + bc0_ref[...]
    h1s = c1s + ws + cs * wn
    rho = dot(h1s, w11_ref[...]) + b11_ref[...] - y1s
    q = dot(wn, w01_ref[...])
    rn = eq * rho + (1.0 - eq) * (cn * q)
    sn = al * q + cn * rho
    o_ref[...] = dot(y1n + rn, ws1_ref[...]) + dot(a1 + sn, wn1_ref[...]) + bc1_ref[...]


def _sc1_body(y0_h, src_h, dst_h, usel_h,
              agg_h, cnt_h,
              acc_sh, cnt_sh,
              src_v, dst_v, idx_v, sidx_v, rows_v,
              idx16_v, sidx16_v, rows16_v, cval_v, cval16_v,
              zbuf_v, zi_v, selb_v):
    c = lax.axis_index("c")
    s = lax.axis_index("s")
    z16f = jnp.zeros((16,), jnp.float32)
    z16i = jnp.zeros((16,), jnp.int32)

    def _zb(r, _):
        for j in range(8):
            zbuf_v[r, pl.ds(j * 16, 16)] = z16f
        return 0
    lax.fori_loop(0, 64, _zb, 0)

    def _zi(i, _):
        zi_v[pl.ds(i * 16, 16)] = z16i
        return 0
    lax.fori_loop(0, 64, _zi, 0)

    pltpu.sync_copy(src_h.at[pl.ds(s * EPT_B, EPT_B)], src_v)
    pltpu.sync_copy(dst_h.at[pl.ds(s * EPT_B, EPT_B)], dst_v)

    @pl.when(c == 0)
    def _agg():
        def _zcp(q_, _):
            pltpu.sync_copy(zbuf_v, acc_sh.at[pl.ds(s * 640 + q_ * 64, 64)])
            return 0
        lax.fori_loop(0, 10, _zcp, 0)
        plsc.subcore_barrier()

        def _batch(g, _):
            for j in range(8):
                b0 = g * 128 + j * 16
                sidx_v[pl.ds(j * 16, 16)] = src_v[pl.ds(b0, 16)]
                idx_v[pl.ds(j * 16, 16)] = dst_v[pl.ds(b0, 16)]
            pltpu.sync_copy(y0_h.at[sidx_v], rows_v)
            pltpu.sync_copy(rows_v, acc_sh.at[idx_v], add=True)
            return 0
        lax.fori_loop(0, EPT_B // 128, _batch, 0)
        sidx16_v[...] = src_v[pl.ds(EPT_B - 16, 16)]
        idx16_v[...] = dst_v[pl.ds(EPT_B - 16, 16)]
        pltpu.sync_copy(y0_h.at[sidx16_v], rows16_v)
        pltpu.sync_copy(rows16_v, acc_sh.at[idx16_v], add=True)

        plsc.subcore_barrier()
        pltpu.sync_copy(acc_sh.at[pl.ds(s * 625, 625)], agg_h.at[pl.ds(s * 625, 625)])

    @pl.when(c == 1)
    def _cnt():
        pltpu.sync_copy(usel_h, selb_v)

        def _zcp(q_, _):
            pltpu.sync_copy(zi_v, cnt_sh.at[pl.ds(s * 5120 + q_ * 1024, 1024)])
            return 0
        lax.fori_loop(0, 5, _zcp, 0)
        plsc.subcore_barrier()

        def _classify(base):
            s16 = src_v[pl.ds(base, 16)]
            d16 = dst_v[pl.ds(base, 16)]
            mk = jnp.zeros((16,), jnp.int32)
            anym = jnp.zeros((16,), jnp.int32)
            for k in range(8):
                m = (s16 == selb_v[k, :]).astype(jnp.int32)
                mk = mk + m * k
                anym = anym + m
            return mk * N + d16, anym

        def _batch(g, _):
            for j in range(8):
                cidx, cval = _classify(g * 128 + j * 16)
                idx_v[pl.ds(j * 16, 16)] = cidx
                cval_v[pl.ds(j * 16, 16)] = cval
            pltpu.sync_copy(cval_v, cnt_sh.at[idx_v], add=True)
            return 0
        lax.fori_loop(0, EPT_B // 128, _batch, 0)
        cidx, cval = _classify(EPT_B - 16)
        idx16_v[...] = cidx
        cval16_v[...] = cval
        pltpu.sync_copy(cval16_v, cnt_sh.at[idx16_v], add=True)

        plsc.subcore_barrier()
        pltpu.sync_copy(cnt_sh.at[pl.ds(s * 5120, 5120)],
                        cnt_h.at[pl.ds(s * 5120, 5120)])


def _sc2_body(y1_h, cnt_h, src_h, dst_h, x_h, y0_h, agg_h, aux_h,
              a1p_h, alp_h, grows_h, gscal_h,
              a1_sh, al_sh,
              src_v, dst_v, lists_v, rows_v, gi_v, aidx_v, cvals_v,
              a1acc_v, aacc_v, aux_v, idx8_v, zi16_v, g_v, gs_v):
    c = lax.axis_index("c")
    s = lax.axis_index("s")
    wid = c * NT + s
    iota = lax.iota(jnp.int32, 16)
    z16f = jnp.zeros((16,), jnp.float32)
    z16i = jnp.zeros((16,), jnp.int32)

    for k in range(8):
        def _zl(i, _, k=k):
            lists_v[k, pl.ds(i * 16, 16)] = z16i
            return 0
        lax.fori_loop(0, LISTCAP // 16, _zl, 0)
        for j in range(8):
            a1acc_v[k, pl.ds(j * 16, 16)] = z16f
        aacc_v[k, :] = z16i
        zi16_v[k, :] = z16i

    pltpu.sync_copy(aux_h, aux_v)
    pltpu.sync_copy(aux_h.at[28, pl.ds(0, 8)], idx8_v)
    pltpu.sync_copy(src_h.at[pl.ds(wid * EPT_D, EPT_D)],
                    src_v.at[pl.ds(0, EPT_D)])
    pltpu.sync_copy(dst_h.at[pl.ds(wid * EPT_D, EPT_D)],
                    dst_v.at[pl.ds(0, EPT_D)])

    @pl.when(s == 0)
    def _zero_shared():
        pltpu.sync_copy(a1acc_v, a1_sh)
        pltpu.sync_copy(zi16_v.at[pl.ds(0, 8)], al_sh)
    plsc.subcore_barrier()

    def _chunk(g, cnts):
        base = g * 16
        s16 = src_v[pl.ds(base, 16)]
        d16 = dst_v[pl.ds(base, 16)]
        valid = (base + iota) < EPT_D
        new = []
        for k in range(8):
            m = (d16 == aux_v[k, :]) & valid
            plsc.store_compressed(lists_v.at[k, pl.ds(cnts[k], 16)], s16, mask=m)
            new.append(cnts[k] + jnp.sum(m.astype(jnp.int32)))
        return tuple(new)
    cnts = lax.fori_loop(0, (EPT_D + 15) // 16, _chunk, (0,) * 8)

    for k in range(8):
        cnt_k = cnts[k]
        sk = aux_v[8 + k, :]
        rep16 = aux_v[16 + k, :]

        def _fbatch(b, aacc16, k=k, cnt_k=cnt_k, sk=sk, rep16=rep16):
            for r in range(8):
                gi_v[pl.ds(r * 16, 16)] = lists_v[k, pl.ds(b * 128 + r * 16, 16)]
            pltpu.sync_copy(y1_h.at[gi_v], rows_v)
            nv = jnp.minimum(cnt_k - b * 128, 128)

            def _accrow(j, _, k=k):
                for rb in range(8):
                    a1acc_v[k, pl.ds(rb * 16, 16)] = (
                        a1acc_v[k, pl.ds(rb * 16, 16)]
                        + rows_v[j, pl.ds(rb * 16, 16)]
                    )
                return 0
            lax.fori_loop(0, nv, _accrow, 0)

            for r in range(8):
                aidx_v[pl.ds(r * 16, 16)] = gi_v[pl.ds(r * 16, 16)] + rep16
            pltpu.sync_copy(cnt_h.at[aidx_v], cvals_v)
            for r in range(8):
                lanes = iota + (b * 128 + r * 16)
                lv = lanes < cnt_k
                srcv = gi_v[pl.ds(r * 16, 16)]
                cv = cvals_v[pl.ds(r * 16, 16)]
                aacc16 = aacc16 + jnp.where(lv & (srcv != sk), cv, 0)
            return aacc16

        nbat = (cnt_k + 127) // 128
        aacc16 = lax.fori_loop(0, nbat, _fbatch, jnp.zeros((16,), jnp.int32))
        aacc_v[k, :] = aacc16

    pltpu.sync_copy(a1acc_v, a1_sh.at[idx8_v], add=True)
    pltpu.sync_copy(aacc_v, al_sh.at[idx8_v], add=True)
    plsc.subcore_barrier()

    @pl.when(s == 0)
    def _writeout():
        pltpu.sync_copy(a1_sh, a1p_h.at[c])
        pltpu.sync_copy(al_sh, alp_h.at[c])

    @pl.when((c == 0) & (s == 1))
    def _gathers():
        pltpu.sync_copy(aux_v.at[24], gs_v)
        pltpu.sync_copy(x_h.at[gs_v], g_v)
        pltpu.sync_copy(g_v, grows_h.at[pl.ds(0, 16)])
        pltpu.sync_copy(y0_h.at[gs_v], g_v)
        pltpu.sync_copy(g_v, grows_h.at[pl.ds(16, 16)])
        pltpu.sync_copy(agg_h.at[gs_v], g_v)
        pltpu.sync_copy(g_v, grows_h.at[pl.ds(32, 16)])
        pltpu.sync_copy(y1_h.at[gs_v], g_v)
        pltpu.sync_copy(g_v, grows_h.at[pl.ds(48, 16)])
        pltpu.sync_copy(aux_v.at[25], gs_v)
        pltpu.sync_copy(y1_h.at[gs_v], g_v)
        pltpu.sync_copy(g_v, grows_h.at[pl.ds(64, 16)])
        pltpu.sync_copy(aux_v.at[26], gs_v)
        pltpu.sync_copy(cnt_h.at[gs_v], gi_v.at[pl.ds(0, 16)])
        pltpu.sync_copy(gi_v.at[pl.ds(0, 16)], gscal_h.at[0])
        pltpu.sync_copy(aux_v.at[27], gs_v)
        pltpu.sync_copy(cnt_h.at[gs_v], gi_v.at[pl.ds(0, 16)])
        pltpu.sync_copy(gi_v.at[pl.ds(0, 16)], gscal_h.at[1])


def _make_sc1():
    mesh = plsc.VectorSubcoreMesh(core_axis_name="c", subcore_axis_name="s")
    return pl.kernel(
        _sc1_body,
        out_type=(
            jax.ShapeDtypeStruct((N, F), jnp.float32),
            jax.ShapeDtypeStruct((CNT_LEN,), jnp.int32),
        ),
        mesh=mesh,
        scratch_types=(
            pltpu.VMEM_SHARED((ACC_ROWS, F), jnp.float32),
            pltpu.VMEM_SHARED((CNT_LEN,), jnp.int32),
            pltpu.VMEM((EPT_B,), jnp.int32),
            pltpu.VMEM((EPT_B,), jnp.int32),
            pltpu.VMEM((128,), jnp.int32),
            pltpu.VMEM((128,), jnp.int32),
            pltpu.VMEM((128, F), jnp.float32),
            pltpu.VMEM((16,), jnp.int32),
            pltpu.VMEM((16,), jnp.int32),
            pltpu.VMEM((16, F), jnp.float32),
            pltpu.VMEM((128,), jnp.int32),
            pltpu.VMEM((16,), jnp.int32),
            pltpu.VMEM((64, 128), jnp.float32),
            pltpu.VMEM((1024,), jnp.int32),
            pltpu.VMEM((8, 16), jnp.int32),
        ),
        name="sc_agg_cnt",
    )


def _make_sc2():
    mesh = plsc.VectorSubcoreMesh(core_axis_name="c", subcore_axis_name="s")
    return pl.kernel(
        _sc2_body,
        out_type=(
            jax.ShapeDtypeStruct((2, 8, F), jnp.float32),
            jax.ShapeDtypeStruct((2, 8, 16), jnp.int32),
            jax.ShapeDtypeStruct((80, F), jnp.float32),
            jax.ShapeDtypeStruct((2, 16), jnp.int32),
        ),
        mesh=mesh,
        scratch_types=(
            pltpu.VMEM_SHARED((8, F), jnp.float32),
            pltpu.VMEM_SHARED((8, 16), jnp.int32),
            pltpu.VMEM((EPT_D + 16,), jnp.int32),
            pltpu.VMEM((EPT_D + 16,), jnp.int32),
            pltpu.VMEM((8, LISTCAP), jnp.int32),
            pltpu.VMEM((128, F), jnp.float32),
            pltpu.VMEM((128,), jnp.int32),
            pltpu.VMEM((128,), jnp.int32),
            pltpu.VMEM((128,), jnp.int32),
            pltpu.VMEM((8, F), jnp.float32),
            pltpu.VMEM((8, 16), jnp.int32),
            pltpu.VMEM((32, 16), jnp.int32),
            pltpu.VMEM((8,), jnp.int32),
            pltpu.VMEM((16, 16), jnp.int32),
            pltpu.VMEM((16, F), jnp.float32),
            pltpu.VMEM((16,), jnp.int32),
        ),
        name="sc_stats_gather",
    )


def kernel(x, params, edge_index, pos):
    f32 = jnp.float32
    src = edge_index[0].astype(jnp.int32)
    dst = edge_index[1].astype(jnp.int32)
    pos = pos.astype(jnp.int32)
    s_ids = pos[:, ::-1].reshape(-1)
    n_ids = pos.reshape(-1)

    w00, b00 = params["f0_0"]
    w10, b10 = params["f1_0"]
    ws0, wn0, bc0 = params["conv_0"]
    w01, b01 = params["f0_1"]
    w11, b11 = params["f1_1"]
    ws1, wn1, bc1 = params["conv_1"]

    rep = jnp.argmax(s_ids[:, None] == s_ids[None, :], axis=1).astype(jnp.int32)
    is_rep = rep == jnp.arange(8, dtype=jnp.int32)
    uniq_sel = jnp.where(is_rep, s_ids, -1)
    usel_b = jnp.broadcast_to(uniq_sel[:, None], (8, 16)).astype(jnp.int32)

    pad8 = lambda a: jnp.concatenate([a, jnp.zeros((8,), jnp.int32)])
    aux = jnp.zeros((32, 16), jnp.int32)
    aux = aux.at[0:8].set(jnp.broadcast_to(n_ids[:, None], (8, 16)))
    aux = aux.at[8:16].set(jnp.broadcast_to(s_ids[:, None], (8, 16)))
    aux = aux.at[16:24].set(jnp.broadcast_to((rep * N)[:, None], (8, 16)))
    aux = aux.at[24].set(pad8(s_ids))
    aux = aux.at[25].set(pad8(n_ids))
    aux = aux.at[26].set(pad8(rep * N + n_ids))
    aux = aux.at[27].set(pad8(rep * N + s_ids))
    aux = aux.at[28].set(jnp.arange(16, dtype=jnp.int32))

    row_spec = pl.BlockSpec((2000, F), lambda i: (i, 0))
    w_spec = pl.BlockSpec((F, F), lambda i: (0, 0))
    b_spec = pl.BlockSpec((1, F), lambda i: (0, 0))

    y0 = pl.pallas_call(
        _mm_bias_body,
        grid=(N // 2000,),
        in_specs=[row_spec, w_spec, b_spec],
        out_specs=row_spec,
        out_shape=jax.ShapeDtypeStruct((N, F), f32),
    )(x, w00, b00.reshape(1, F))

    agg0, cnt = _make_sc1()(y0, src, dst, usel_b)

    y1 = pl.pallas_call(
        _fused_c_body,
        grid=(N // 2000,),
        in_specs=[row_spec, row_spec, w_spec, w_spec, b_spec, w_spec, b_spec],
        out_specs=row_spec,
        out_shape=jax.ShapeDtypeStruct((N, F), f32),
    )(y0, agg0, ws0, wn0, bc0.reshape(1, F), w01, b01.reshape(1, F))

    a1p, alp, grows, gscal = _make_sc2()(y1, cnt, src, dst, x, y0, agg0, aux)

    bcast = lambda v: jnp.broadcast_to(v.astype(f32)[:, None], (8, F))
    cn8 = bcast(gscal[0, :8])
    cs8 = bcast(gscal[1, :8])
    al8 = bcast(alp.sum(axis=(0, 2)))
    eq8 = bcast((s_ids == n_ids).astype(jnp.int32))

    full = lambda s_: pl.BlockSpec(s_, lambda: tuple(0 for _ in s_))
    out8 = pl.pallas_call(
        _final_body,
        in_specs=[full((80, F)), full((2, 8, F))] + [full((8, F))] * 4
        + [full((F, F)), full((1, F)), full((F, F)), full((F, F)), full((1, F)),
           full((F, F)), full((1, F)), full((F, F)), full((1, F)),
           full((F, F)), full((F, F)), full((1, F))],
        out_specs=full((8, F)),
        out_shape=jax.ShapeDtypeStruct((8, F), f32),
    )(grows, a1p, cn8, cs8, al8, eq8,
      w10, b10.reshape(1, F), ws0, wn0, bc0.reshape(1, F), w01, b01.reshape(1, F),
      w11, b11.reshape(1, F), ws1, wn1, bc1.reshape(1, F))

    return out8.reshape(pos.shape[0], 2, F)


# speedup vs baseline: 29.7518x; 29.7518x over previous
import jax
import jax.numpy as jnp
from jax import lax
from jax.experimental import pallas as pl
from jax.experimental.pallas import tpu as pltpu
from jax.experimental.pallas import tpu_sc as plsc

N = 10000
E = 160000
F = 128
NT = 16
EPT_B = E // NT
EPT_D = E // 32
ACC_ROWS = 10112
CNT_LEN = 81920
LISTCAP = EPT_D + 16


def _mm_bias_body(x_ref, w_ref, b_ref, o_ref):
    o_ref[...] = (
        jnp.dot(x_ref[...], w_ref[...], preferred_element_type=jnp.float32)
        + b_ref[...]
    )


def _fused_c_body(y0_ref, agg_ref, ws_ref, wn_ref, bc_ref, w01_ref, b01_ref, o_ref):
    c1 = (
        jnp.dot(y0_ref[...], ws_ref[...], preferred_element_type=jnp.float32)
        + jnp.dot(agg_ref[...], wn_ref[...], preferred_element_type=jnp.float32)
        + bc_ref[...]
    )
    o_ref[...] = (
        jnp.dot(c1, w01_ref[...], preferred_element_type=jnp.float32) + b01_ref[...]
    )


def _final_body(grows_ref, a1_ref, cn_ref, cs_ref, al_ref, eq_ref,
                w10_ref, b10_ref, ws0_ref, wn0_ref, bc0_ref, w01_ref, b01_ref,
                w11_ref, b11_ref, ws1_ref, wn1_ref, bc1_ref, o_ref):
    dot = lambda a, b: jnp.dot(a, b, preferred_element_type=jnp.float32)
    g = grows_ref[...]
    xs, y0s, aggs, y1s, y1n = g[0:8], g[16:24], g[32:40], g[48:56], g[64:72]
    a1 = a1_ref[...]
    cn, cs, al, eq = cn_ref[...], cs_ref[...], al_ref[...], eq_ref[...]
    d = dot(xs, w10_ref[...]) + b10_ref[...] - y0s
    ws = dot(d, ws0_ref[...])
    wn = dot(d, wn0_ref[...])
    c1s = dot(y0s, ws0_ref[...]) + dot(aggs, wn0_ref[...]) + bc0_ref[...]
    h1s = c1s + ws + cs * wn
    rho = dot(h1s, w11_ref[...]) + b11_ref[...] - y1s
    q = dot(wn, w01_ref[...])
    rn = eq * rho + (1.0 - eq) * (cn * q)
    alpha = al - cn * cs
    sn = alpha * q + cn * rho
    o_ref[...] = dot(y1n + rn, ws1_ref[...]) + dot(a1 + sn, wn1_ref[...]) + bc1_ref[...]


def _sc1_body(y0_h, src_h, dst_h, usel_h,
              agg_h, cnt_h,
              acc_sh, cnt_sh,
              src_v, dst_v, idx_v, sidx_v, rows_v,
              idx16_v, sidx16_v, rows16_v, cval_v, cval16_v, selb_v):
    c = lax.axis_index("c")
    s = lax.axis_index("s")
    z16f = jnp.zeros((16,), jnp.float32)
    z16i = jnp.zeros((16,), jnp.int32)

    @pl.when(c == 0)
    def _agg():
        def _zb(r, _):
            for j in range(8):
                rows_v[r, pl.ds(j * 16, 16)] = z16f
            return 0
        lax.fori_loop(0, 128, _zb, 0)

        def _zcp(q_, _):
            pltpu.sync_copy(rows_v.at[pl.ds(0, 64)],
                            acc_sh.at[pl.ds(s * 632 + q_ * 64, 64)])
            return 0
        lax.fori_loop(0, 9, _zcp, 0)
        pltpu.sync_copy(rows_v.at[pl.ds(0, 56)],
                        acc_sh.at[pl.ds(s * 632 + 576, 56)])
        pltpu.sync_copy(src_h.at[pl.ds(s * EPT_B, EPT_B)], src_v)
        pltpu.sync_copy(dst_h.at[pl.ds(s * EPT_B, EPT_B)], dst_v)
        plsc.subcore_barrier()

        def _batch(g, _):
            for j in range(8):
                b0 = g * 128 + j * 16
                sidx_v[pl.ds(j * 16, 16)] = src_v[pl.ds(b0, 16)]
                idx_v[pl.ds(j * 16, 16)] = dst_v[pl.ds(b0, 16)]
            pltpu.sync_copy(y0_h.at[sidx_v], rows_v)
            pltpu.sync_copy(rows_v, acc_sh.at[idx_v], add=True)
            return 0
        lax.fori_loop(0, EPT_B // 128, _batch, 0)
        sidx16_v[...] = src_v[pl.ds(EPT_B - 16, 16)]
        idx16_v[...] = dst_v[pl.ds(EPT_B - 16, 16)]
        pltpu.sync_copy(y0_h.at[sidx16_v], rows16_v)
        pltpu.sync_copy(rows16_v, acc_sh.at[idx16_v], add=True)

        plsc.subcore_barrier()
        pltpu.sync_copy(acc_sh.at[pl.ds(s * 624, 624)], agg_h.at[pl.ds(s * 624, 624)])

        @pl.when(s == NT - 1)
        def _tail():
            pltpu.sync_copy(acc_sh.at[pl.ds(9984, 16)], agg_h.at[pl.ds(9984, 16)])

    @pl.when(c == 1)
    def _cnt():
        pltpu.sync_copy(usel_h, selb_v)

        def _zsrc(i, _):
            src_v[pl.ds(i * 16, 16)] = z16i
            return 0
        lax.fori_loop(0, 64, _zsrc, 0)

        def _zcp(q_, _):
            pltpu.sync_copy(src_v.at[pl.ds(0, 1024)],
                            cnt_sh.at[pl.ds(s * 5120 + q_ * 1024, 1024)])
            return 0
        lax.fori_loop(0, 5, _zcp, 0)
        pltpu.sync_copy(src_h.at[pl.ds(s * EPT_B, EPT_B)], src_v)
        pltpu.sync_copy(dst_h.at[pl.ds(s * EPT_B, EPT_B)], dst_v)
        plsc.subcore_barrier()

        one16 = jnp.ones((16,), jnp.int32)

        def _classify(base):
            s16 = src_v[pl.ds(base, 16)]
            d16 = dst_v[pl.ds(base, 16)]
            mk = jnp.zeros((16,), jnp.int32)
            anym = jnp.zeros((16,), jnp.int32)
            for k in range(8):
                m = jnp.where(s16 == selb_v[k, :], one16, z16i)
                mk = mk + m * k
                anym = anym + m
            return mk * N + d16, anym

        def _batch(g, _):
            for j in range(8):
                cidx, cval = _classify(g * 128 + j * 16)
                idx_v[pl.ds(j * 16, 16)] = cidx
                cval_v[pl.ds(j * 16, 16)] = cval
            pltpu.sync_copy(cval_v, cnt_sh.at[idx_v], add=True)
            return 0
        lax.fori_loop(0, EPT_B // 128, _batch, 0)
        cidx, cval = _classify(EPT_B - 16)
        idx16_v[...] = cidx
        cval16_v[...] = cval
        pltpu.sync_copy(cval16_v, cnt_sh.at[idx16_v], add=True)

        plsc.subcore_barrier()
        pltpu.sync_copy(cnt_sh.at[pl.ds(s * 5120, 5120)],
                        cnt_h.at[pl.ds(s * 5120, 5120)])


def _sc2a_body(y1_h, src_h, dst_h,
               agg1_h,
               acc_sh,
               src_v, dst_v, idx_v, sidx_v, rows_v,
               idx16_v, sidx16_v, rows16_v):
    c = lax.axis_index("c")
    s = lax.axis_index("s")
    z16f = jnp.zeros((16,), jnp.float32)

    @pl.when(c == 0)
    def _agg1():
        def _zb(r, _):
            for j in range(8):
                rows_v[r, pl.ds(j * 16, 16)] = z16f
            return 0
        lax.fori_loop(0, 64, _zb, 0)

        def _zcp(q_, _):
            pltpu.sync_copy(rows_v.at[pl.ds(0, 64)],
                            acc_sh.at[pl.ds(s * 632 + q_ * 64, 64)])
            return 0
        lax.fori_loop(0, 9, _zcp, 0)
        pltpu.sync_copy(rows_v.at[pl.ds(0, 56)],
                        acc_sh.at[pl.ds(s * 632 + 576, 56)])
        pltpu.sync_copy(src_h.at[pl.ds(s * EPT_B, EPT_B)],
                        src_v.at[pl.ds(0, EPT_B)])
        pltpu.sync_copy(dst_h.at[pl.ds(s * EPT_B, EPT_B)],
                        dst_v.at[pl.ds(0, EPT_B)])
        plsc.subcore_barrier()

        def _batch(g, _):
            for j in range(4):
                b0 = g * 64 + j * 16
                sidx_v[pl.ds(j * 16, 16)] = src_v[pl.ds(b0, 16)]
                idx_v[pl.ds(j * 16, 16)] = dst_v[pl.ds(b0, 16)]
            pltpu.sync_copy(y1_h.at[sidx_v], rows_v)
            pltpu.sync_copy(rows_v, acc_sh.at[idx_v], add=True)
            return 0
        lax.fori_loop(0, EPT_B // 64, _batch, 0)
        sidx16_v[...] = src_v[pl.ds(EPT_B - 16, 16)]
        idx16_v[...] = dst_v[pl.ds(EPT_B - 16, 16)]
        pltpu.sync_copy(y1_h.at[sidx16_v], rows16_v)
        pltpu.sync_copy(rows16_v, acc_sh.at[idx16_v], add=True)

        plsc.subcore_barrier()
        pltpu.sync_copy(acc_sh.at[pl.ds(s * 624, 624)],
                        agg1_h.at[pl.ds(s * 624, 624)])

        @pl.when(s == NT - 1)
        def _tail():
            pltpu.sync_copy(acc_sh.at[pl.ds(9984, 16)], agg1_h.at[pl.ds(9984, 16)])


def _make_sc2a():
    mesh = plsc.VectorSubcoreMesh(core_axis_name="c", subcore_axis_name="s")
    return pl.kernel(
        _sc2a_body,
        out_type=(jax.ShapeDtypeStruct((N, F), jnp.float32),),
        mesh=mesh,
        scratch_types=(
            pltpu.VMEM_SHARED((ACC_ROWS, F), jnp.float32),
            pltpu.VMEM((EPT_B,), jnp.int32),
            pltpu.VMEM((EPT_B,), jnp.int32),
            pltpu.VMEM((64,), jnp.int32),
            pltpu.VMEM((64,), jnp.int32),
            pltpu.VMEM((64, F), jnp.float32),
            pltpu.VMEM((16,), jnp.int32),
            pltpu.VMEM((16,), jnp.int32),
            pltpu.VMEM((16, F), jnp.float32),
        ),
        name="sc_agg1",
    )


def _sc2b_body(y1_h, cnt_h, src_h, dst_h, x_h, y0_h, agg_h, aux_h,
               alp_h, grows_h, gscal_h,
               al_sh,
               cnt_v, src_v, dst_v, aacc_v, aux_v, idtile_v, zi16_v,
               g_v, gs_v, idx16_v):
    c = lax.axis_index("c")
    s = lax.axis_index("s")
    wid = c * NT + s
    iota = lax.iota(jnp.int32, 16)
    z16i = jnp.zeros((16,), jnp.int32)

    pltpu.sync_copy(aux_h, aux_v)
    for k in range(16):
        aacc_v[k, :] = z16i
        zi16_v[k, :] = z16i
    idtile_v[...] = iota

    @pl.when(s == 0)
    def _zero_shared():
        pltpu.sync_copy(zi16_v, al_sh)

    pltpu.sync_copy(cnt_h.at[pl.ds(0, 80128)], cnt_v)
    pltpu.sync_copy(src_h.at[pl.ds(wid * EPT_D, EPT_D)],
                    src_v.at[pl.ds(0, EPT_D)])
    pltpu.sync_copy(dst_h.at[pl.ds(wid * EPT_D, EPT_D)],
                    dst_v.at[pl.ds(0, EPT_D)])
    plsc.subcore_barrier()

    nfull = [aux_v[k, :] for k in range(8)]
    srep = [aux_v[8 + k, :] for k in range(8)]

    def _chunk(g, _):
        base = g * 16
        s16 = src_v[pl.ds(base, 16)]
        d16 = dst_v[pl.ds(base, 16)]
        valid = (base + iota) < EPT_D
        for k in range(8):
            m = (d16 == nfull[k]) & valid
            aidx = jnp.where(valid, s16 + srep[k], z16i)
            cv = plsc.load_gather(cnt_v, [aidx])
            aacc_v[k, :] = aacc_v[k, :] + jnp.where(m, cv, z16i)
        return 0
    lax.fori_loop(0, (EPT_D + 15) // 16, _chunk, 0)

    pltpu.sync_copy(aacc_v, al_sh.at[idtile_v], add=True)
    plsc.subcore_barrier()

    @pl.when(s == 0)
    def _writeout():
        pltpu.sync_copy(al_sh.at[pl.ds(0, 8)], alp_h.at[c])

    @pl.when((c == 1) & (s == 1))
    def _gathers():
        gs_v[...] = aux_v[24, :]
        pltpu.sync_copy(x_h.at[gs_v], g_v)
        pltpu.sync_copy(g_v, grows_h.at[pl.ds(0, 16)])
        pltpu.sync_copy(y0_h.at[gs_v], g_v)
        pltpu.sync_copy(g_v, grows_h.at[pl.ds(16, 16)])
        pltpu.sync_copy(agg_h.at[gs_v], g_v)
        pltpu.sync_copy(g_v, grows_h.at[pl.ds(32, 16)])
        pltpu.sync_copy(y1_h.at[gs_v], g_v)
        pltpu.sync_copy(g_v, grows_h.at[pl.ds(48, 16)])
        gs_v[...] = aux_v[25, :]
        pltpu.sync_copy(y1_h.at[gs_v], g_v)
        pltpu.sync_copy(g_v, grows_h.at[pl.ds(64, 16)])
        gs_v[...] = aux_v[26, :]
        pltpu.sync_copy(cnt_h.at[gs_v], idx16_v)
        pltpu.sync_copy(idx16_v, gscal_h.at[pl.ds(0, 16)])
        gs_v[...] = aux_v[27, :]
        pltpu.sync_copy(cnt_h.at[gs_v], idx16_v)
        pltpu.sync_copy(idx16_v, gscal_h.at[pl.ds(16, 16)])


def _make_sc2b():
    mesh = plsc.VectorSubcoreMesh(core_axis_name="c", subcore_axis_name="s")
    return pl.kernel(
        _sc2b_body,
        out_type=(
            jax.ShapeDtypeStruct((2, 8, 16), jnp.int32),
            jax.ShapeDtypeStruct((80, F), jnp.float32),
            jax.ShapeDtypeStruct((32,), jnp.int32),
        ),
        mesh=mesh,
        compiler_params=pltpu.CompilerParams(needs_layout_passes=False),
        scratch_types=(
            pltpu.VMEM_SHARED((16, 16), jnp.int32),
            pltpu.VMEM((80128,), jnp.int32),
            pltpu.VMEM((EPT_D + 16,), jnp.int32),
            pltpu.VMEM((EPT_D + 16,), jnp.int32),
            pltpu.VMEM((16, 16), jnp.int32),
            pltpu.VMEM((32, 16), jnp.int32),
            pltpu.VMEM((16,), jnp.int32),
            pltpu.VMEM((16, 16), jnp.int32),
            pltpu.VMEM((16, F), jnp.float32),
            pltpu.VMEM((16,), jnp.int32),
            pltpu.VMEM((16,), jnp.int32),
        ),
        name="sc_beta_gather",
    )


def _make_sc1():
    mesh = plsc.VectorSubcoreMesh(core_axis_name="c", subcore_axis_name="s")
    return pl.kernel(
        _sc1_body,
        out_type=(
            jax.ShapeDtypeStruct((N, F), jnp.float32),
            jax.ShapeDtypeStruct((CNT_LEN,), jnp.int32),
        ),
        mesh=mesh,
        scratch_types=(
            pltpu.VMEM_SHARED((ACC_ROWS, F), jnp.float32),
            pltpu.VMEM_SHARED((CNT_LEN,), jnp.int32),
            pltpu.VMEM((EPT_B,), jnp.int32),
            pltpu.VMEM((EPT_B,), jnp.int32),
            pltpu.VMEM((128,), jnp.int32),
            pltpu.VMEM((128,), jnp.int32),
            pltpu.VMEM((128, F), jnp.float32),
            pltpu.VMEM((16,), jnp.int32),
            pltpu.VMEM((16,), jnp.int32),
            pltpu.VMEM((16, F), jnp.float32),
            pltpu.VMEM((128,), jnp.int32),
            pltpu.VMEM((16,), jnp.int32),
            pltpu.VMEM((8, 16), jnp.int32),
        ),
        name="sc_agg_cnt",
    )


def kernel(x, params, edge_index, pos):
    f32 = jnp.float32
    src = edge_index[0].astype(jnp.int32)
    dst = edge_index[1].astype(jnp.int32)
    pos = pos.astype(jnp.int32)
    s_ids = pos[:, ::-1].reshape(-1)
    n_ids = pos.reshape(-1)

    w00, b00 = params["f0_0"]
    w10, b10 = params["f1_0"]
    ws0, wn0, bc0 = params["conv_0"]
    w01, b01 = params["f0_1"]
    w11, b11 = params["f1_1"]
    ws1, wn1, bc1 = params["conv_1"]

    rep = jnp.argmax(s_ids[:, None] == s_ids[None, :], axis=1).astype(jnp.int32)
    is_rep = rep == jnp.arange(8, dtype=jnp.int32)
    uniq_sel = jnp.where(is_rep, s_ids, -1)
    usel_b = jnp.broadcast_to(uniq_sel[:, None], (8, 16)).astype(jnp.int32)

    pad8 = lambda a: jnp.concatenate([a, jnp.zeros((8,), jnp.int32)])
    aux = jnp.zeros((32, 16), jnp.int32)
    aux = aux.at[0:8].set(jnp.broadcast_to(n_ids[:, None], (8, 16)))
    aux = aux.at[8:16].set(jnp.broadcast_to((rep * N)[:, None], (8, 16)))
    aux = aux.at[24].set(pad8(s_ids))
    aux = aux.at[25].set(pad8(n_ids))
    aux = aux.at[26].set(pad8(rep * N + n_ids))
    aux = aux.at[27].set(pad8(rep * N + s_ids))

    row_spec = pl.BlockSpec((2000, F), lambda i: (i, 0))
    w_spec = pl.BlockSpec((F, F), lambda i: (0, 0))
    b_spec = pl.BlockSpec((1, F), lambda i: (0, 0))

    y0 = pl.pallas_call(
        _mm_bias_body,
        grid=(N // 2000,),
        in_specs=[row_spec, w_spec, b_spec],
        out_specs=row_spec,
        out_shape=jax.ShapeDtypeStruct((N, F), f32),
    )(x, w00, b00.reshape(1, F))

    agg0, cnt = _make_sc1()(y0, src, dst, usel_b)

    y1 = pl.pallas_call(
        _fused_c_body,
        grid=(N // 2000,),
        in_specs=[row_spec, row_spec, w_spec, w_spec, b_spec, w_spec, b_spec],
        out_specs=row_spec,
        out_shape=jax.ShapeDtypeStruct((N, F), f32),
    )(y0, agg0, ws0, wn0, bc0.reshape(1, F), w01, b01.reshape(1, F))

    agg1, = _make_sc2a()(y1, src, dst)
    alp, grows, gscal = _make_sc2b()(y1, cnt, src, dst, x, y0, agg0, aux)
    a1 = agg1[n_ids]

    bcast = lambda v: jnp.broadcast_to(v.astype(f32)[:, None], (8, F))
    cn8 = bcast(gscal[0:8])
    cs8 = bcast(gscal[16:24])
    al8 = bcast(alp.sum(axis=(0, 2)))
    eq8 = bcast((s_ids == n_ids).astype(jnp.int32))

    full = lambda s_: pl.BlockSpec(s_, lambda: tuple(0 for _ in s_))
    out8 = pl.pallas_call(
        _final_body,
        in_specs=[full((80, F))] + [full((8, F))] * 5
        + [full((F, F)), full((1, F)), full((F, F)), full((F, F)), full((1, F)),
           full((F, F)), full((1, F)), full((F, F)), full((1, F)),
           full((F, F)), full((F, F)), full((1, F))],
        out_specs=full((8, F)),
        out_shape=jax.ShapeDtypeStruct((8, F), f32),
    )(grows, a1, cn8, cs8, al8, eq8,
      w10, b10.reshape(1, F), ws0, wn0, bc0.reshape(1, F), w01, b01.reshape(1, F),
      w11, b11.reshape(1, F), ws1, wn1, bc1.reshape(1, F))

    return out8.reshape(pos.shape[0], 2, F)
